# baseline (device time: 155453 ns/iter reference)
import jax
import jax.numpy as jnp
from jax import lax
from jax.experimental import pallas as pl
from jax.experimental.pallas import tpu as pltpu

N_DEV = 4


def kernel(x, w_mat):
    m_global, k_per = x.shape
    _, n = w_mat.shape
    m_per = m_global // N_DEV

    def body(x_ref, w_ref, out_ref, comm_ref, send_sems, recv_sems):
        my = lax.axis_index("i")
        left = (my - 1) % N_DEV
        right = (my + 1) % N_DEV

        barrier_sem = pltpu.get_barrier_semaphore()
        for nbr in [left, right]:
            pl.semaphore_signal(
                barrier_sem, inc=1,
                device_id=(nbr,), device_id_type=pl.DeviceIdType.MESH,
            )
        pl.semaphore_wait(barrier_sem, 2)

        def partial_chunk(c):
            return jnp.dot(
                x_ref[pl.ds(c * m_per, m_per), :],
                w_ref[:, :],
                preferred_element_type=jnp.float32,
            )

        comm_ref[0, :, :] = partial_chunk((my - 1) % N_DEV)

        for s in range(N_DEV - 1):
            rdma = pltpu.make_async_remote_copy(
                src_ref=comm_ref.at[s],
                dst_ref=comm_ref.at[s + 1],
                send_sem=send_sems.at[s],
                recv_sem=recv_sems.at[s],
                device_id=(right,),
                device_id_type=pl.DeviceIdType.MESH,
            )
            rdma.start()
            rdma.wait()
            c = (my - s - 2) % N_DEV
            comm_ref[s + 1, :, :] += partial_chunk(c)

        y = comm_ref[N_DEV - 1, :, :]
        out_ref[:, :] = y * jax.nn.sigmoid(y)

    return pl.pallas_call(
        body,
        out_shape=jax.ShapeDtypeStruct((m_per, n), jnp.float32),
        in_specs=[
            pl.BlockSpec(memory_space=pltpu.VMEM),
            pl.BlockSpec(memory_space=pltpu.VMEM),
        ],
        out_specs=pl.BlockSpec(memory_space=pltpu.VMEM),
        scratch_shapes=[
            pltpu.VMEM((N_DEV, m_per, n), jnp.float32),
            pltpu.SemaphoreType.DMA((N_DEV - 1,)),
            pltpu.SemaphoreType.DMA((N_DEV - 1,)),
        ],
        compiler_params=pltpu.CompilerParams(collective_id=0),
    )(x, w_mat)


# device time: 85929 ns/iter; 1.8091x vs baseline; 1.8091x over previous
import jax
import jax.numpy as jnp
from jax import lax
from jax.experimental import pallas as pl
from jax.experimental.pallas import tpu as pltpu

N_DEV = 4


def kernel(x, w_mat):
    m_global, k_per = x.shape
    _, n = w_mat.shape
    m_per = m_global // N_DEV
    n_half = n // 2

    def body(x_ref, w_ref, out_ref,
             comm_r_ref, comm_l_ref,
             send_sems_r, recv_sems_r, send_sems_l, recv_sems_l):
        my = lax.axis_index("i")
        left = (my - 1) % N_DEV
        right = (my + 1) % N_DEV

        barrier_sem = pltpu.get_barrier_semaphore()
        for nbr in [left, right]:
            pl.semaphore_signal(
                barrier_sem, inc=1,
                device_id=(nbr,), device_id_type=pl.DeviceIdType.MESH,
            )
        pl.semaphore_wait(barrier_sem, 2)

        def partial_r(c):
            return jnp.dot(
                x_ref[pl.ds(c * m_per, m_per), :], w_ref[:, :n_half],
                preferred_element_type=jnp.float32,
            )

        def partial_l(c):
            return jnp.dot(
                x_ref[pl.ds(c * m_per, m_per), :], w_ref[:, n_half:],
                preferred_element_type=jnp.float32,
            )

        comm_r_ref[0, :, :] = partial_r((my - 1) % N_DEV)
        comm_l_ref[0, :, :] = partial_l((my + 1) % N_DEV)

        for s in range(N_DEV - 1):
            rdma_r = pltpu.make_async_remote_copy(
                src_ref=comm_r_ref.at[s],
                dst_ref=comm_r_ref.at[s + 1],
                send_sem=send_sems_r.at[s],
                recv_sem=recv_sems_r.at[s],
                device_id=(right,),
                device_id_type=pl.DeviceIdType.MESH,
            )
            rdma_l = pltpu.make_async_remote_copy(
                src_ref=comm_l_ref.at[s],
                dst_ref=comm_l_ref.at[s + 1],
                send_sem=send_sems_l.at[s],
                recv_sem=recv_sems_l.at[s],
                device_id=(left,),
                device_id_type=pl.DeviceIdType.MESH,
            )
            rdma_r.start()
            rdma_l.start()

            acc_r = partial_r((my - s - 2) % N_DEV)
            acc_l = partial_l((my + s + 2) % N_DEV)

            rdma_r.wait_recv()
            comm_r_ref[s + 1, :, :] += acc_r
            rdma_l.wait_recv()
            comm_l_ref[s + 1, :, :] += acc_l
            rdma_r.wait_send()
            rdma_l.wait_send()

        yr = comm_r_ref[N_DEV - 1, :, :]
        out_ref[:, :n_half] = yr * jax.nn.sigmoid(yr)
        yl = comm_l_ref[N_DEV - 1, :, :]
        out_ref[:, n_half:] = yl * jax.nn.sigmoid(yl)

    return pl.pallas_call(
        body,
        out_shape=jax.ShapeDtypeStruct((m_per, n), jnp.float32),
        in_specs=[
            pl.BlockSpec(memory_space=pltpu.VMEM),
            pl.BlockSpec(memory_space=pltpu.VMEM),
        ],
        out_specs=pl.BlockSpec(memory_space=pltpu.VMEM),
        scratch_shapes=[
            pltpu.VMEM((N_DEV, m_per, n_half), jnp.float32),
            pltpu.VMEM((N_DEV, m_per, n_half), jnp.float32),
            pltpu.SemaphoreType.DMA((N_DEV - 1,)),
            pltpu.SemaphoreType.DMA((N_DEV - 1,)),
            pltpu.SemaphoreType.DMA((N_DEV - 1,)),
            pltpu.SemaphoreType.DMA((N_DEV - 1,)),
        ],
        compiler_params=pltpu.CompilerParams(collective_id=0),
    )(x, w_mat)


# device time: 80520 ns/iter; 1.9306x vs baseline; 1.0672x over previous
import jax
import jax.numpy as jnp
from jax import lax
from jax.experimental import pallas as pl
from jax.experimental.pallas import tpu as pltpu

N_DEV = 4
SUB = 2


def kernel(x, w_mat):
    m_global, k_per = x.shape
    _, n = w_mat.shape
    m_per = m_global // N_DEV
    n_sub = n // (2 * SUB)

    def body(x_ref, w_ref, out_ref,
             comm_r_ref, comm_l_ref,
             send_sems_r, recv_sems_r, send_sems_l, recv_sems_l):
        my = lax.axis_index("i")
        left = (my - 1) % N_DEV
        right = (my + 1) % N_DEV

        barrier_sem = pltpu.get_barrier_semaphore()
        for nbr in [left, right]:
            pl.semaphore_signal(
                barrier_sem, inc=1,
                device_id=(nbr,), device_id_type=pl.DeviceIdType.MESH,
            )
        pl.semaphore_wait(barrier_sem, 2)

        def partial(c, col0):
            return jnp.dot(
                x_ref[pl.ds(c * m_per, m_per), :],
                w_ref[:, col0:col0 + n_sub],
                preferred_element_type=jnp.float32,
            )

        rings = []
        for b in range(SUB):
            rings.append((comm_r_ref, send_sems_r, recv_sems_r,
                          right, -1, b, b * n_sub))
            rings.append((comm_l_ref, send_sems_l, recv_sems_l,
                          left, +1, b, n // 2 + b * n_sub))
        rings = [rings[0], rings[1], rings[2], rings[3]]

        def make_rdma(comm, ss, rs, dev, b, s):
            return pltpu.make_async_remote_copy(
                src_ref=comm.at[b, s],
                dst_ref=comm.at[b, s + 1],
                send_sem=ss.at[b, s],
                recv_sem=rs.at[b, s],
                device_id=(dev,),
                device_id_type=pl.DeviceIdType.MESH,
            )

        sends = {}
        for ri, (comm, ss, rs, dev, sign, b, col0) in enumerate(rings):
            comm[b, 0, :, :] = partial((my + sign) % N_DEV, col0)
            rdma = make_rdma(comm, ss, rs, dev, b, 0)
            rdma.start()
            sends[(ri, 0)] = rdma

        for s in range(1, N_DEV):
            for ri, (comm, ss, rs, dev, sign, b, col0) in enumerate(rings):
                acc = partial((my + sign * (1 + s)) % N_DEV, col0)
                sends[(ri, s - 1)].wait_recv()
                comm[b, s, :, :] += acc
                if s < N_DEV - 1:
                    rdma = make_rdma(comm, ss, rs, dev, b, s)
                    rdma.start()
                    sends[(ri, s)] = rdma
                else:
                    y = comm[b, s, :, :]
                    out_ref[:, col0:col0 + n_sub] = y * jax.nn.sigmoid(y)

        for rdma in sends.values():
            rdma.wait_send()

    return pl.pallas_call(
        body,
        out_shape=jax.ShapeDtypeStruct((m_per, n), jnp.float32),
        in_specs=[
            pl.BlockSpec(memory_space=pltpu.VMEM),
            pl.BlockSpec(memory_space=pltpu.VMEM),
        ],
        out_specs=pl.BlockSpec(memory_space=pltpu.VMEM),
        scratch_shapes=[
            pltpu.VMEM((SUB, N_DEV, m_per, n_sub), jnp.float32),
            pltpu.VMEM((SUB, N_DEV, m_per, n_sub), jnp.float32),
            pltpu.SemaphoreType.DMA((SUB, N_DEV - 1)),
            pltpu.SemaphoreType.DMA((SUB, N_DEV - 1)),
            pltpu.SemaphoreType.DMA((SUB, N_DEV - 1)),
            pltpu.SemaphoreType.DMA((SUB, N_DEV - 1)),
        ],
        compiler_params=pltpu.CompilerParams(collective_id=0),
    )(x, w_mat)


# device time: 80070 ns/iter; 1.9415x vs baseline; 1.0056x over previous
import jax
import jax.numpy as jnp
from jax import lax
from jax.experimental import pallas as pl
from jax.experimental.pallas import tpu as pltpu

N_DEV = 4
SUB = 4


def kernel(x, w_mat):
    m_global, k_per = x.shape
    _, n = w_mat.shape
    m_per = m_global // N_DEV
    n_sub = n // (2 * SUB)

    def body(x_ref, w_ref, out_ref,
             comm_r_ref, comm_l_ref,
             send_sems_r, recv_sems_r, send_sems_l, recv_sems_l):
        my = lax.axis_index("i")
        left = (my - 1) % N_DEV
        right = (my + 1) % N_DEV

        barrier_sem = pltpu.get_barrier_semaphore()
        for nbr in [left, right]:
            pl.semaphore_signal(
                barrier_sem, inc=1,
                device_id=(nbr,), device_id_type=pl.DeviceIdType.MESH,
            )
        pl.semaphore_wait(barrier_sem, 2)

        def partial(c, col0):
            return jnp.dot(
                x_ref[pl.ds(c * m_per, m_per), :],
                w_ref[:, col0:col0 + n_sub],
                preferred_element_type=jnp.float32,
            )

        rings = []
        for b in range(SUB):
            rings.append((comm_r_ref, send_sems_r, recv_sems_r,
                          right, -1, b, b * n_sub))
            rings.append((comm_l_ref, send_sems_l, recv_sems_l,
                          left, +1, b, n // 2 + b * n_sub))

        def make_rdma(comm, ss, rs, dev, b, s):
            return pltpu.make_async_remote_copy(
                src_ref=comm.at[b, s],
                dst_ref=comm.at[b, s + 1],
                send_sem=ss.at[b, s],
                recv_sem=rs.at[b, s],
                device_id=(dev,),
                device_id_type=pl.DeviceIdType.MESH,
            )

        sends = {}
        for ri, (comm, ss, rs, dev, sign, b, col0) in enumerate(rings):
            comm[b, 0, :, :] = partial((my + sign) % N_DEV, col0)
            rdma = make_rdma(comm, ss, rs, dev, b, 0)
            rdma.start()
            sends[(ri, 0)] = rdma

        for s in range(1, N_DEV):
            for ri, (comm, ss, rs, dev, sign, b, col0) in enumerate(rings):
                acc = partial((my + sign * (1 + s)) % N_DEV, col0)
                sends[(ri, s - 1)].wait_recv()
                if s < N_DEV - 1:
                    comm[b, s, :, :] += acc
                    rdma = make_rdma(comm, ss, rs, dev, b, s)
                    rdma.start()
                    sends[(ri, s)] = rdma
                else:
                    y = comm[b, s, :, :] + acc
                    out_ref[:, col0:col0 + n_sub] = y * jax.nn.sigmoid(y)

        for rdma in sends.values():
            rdma.wait_send()

    return pl.pallas_call(
        body,
        out_shape=jax.ShapeDtypeStruct((m_per, n), jnp.float32),
        in_specs=[
            pl.BlockSpec(memory_space=pltpu.VMEM),
            pl.BlockSpec(memory_space=pltpu.VMEM),
        ],
        out_specs=pl.BlockSpec(memory_space=pltpu.VMEM),
        scratch_shapes=[
            pltpu.VMEM((SUB, N_DEV, m_per, n_sub), jnp.float32),
            pltpu.VMEM((SUB, N_DEV, m_per, n_sub), jnp.float32),
            pltpu.SemaphoreType.DMA((SUB, N_DEV - 1)),
            pltpu.SemaphoreType.DMA((SUB, N_DEV - 1)),
            pltpu.SemaphoreType.DMA((SUB, N_DEV - 1)),
            pltpu.SemaphoreType.DMA((SUB, N_DEV - 1)),
        ],
        compiler_params=pltpu.CompilerParams(collective_id=0),
    )(x, w_mat)
